# baseline (speedup 1.0000x reference)
import functools

import jax
import jax.numpy as jnp
from jax import lax
from jax.experimental import pallas as pl
from jax.experimental.pallas import tpu as pltpu
from jax.experimental.pallas import tpu_sc as plsc

B = 16384
ITEM_VOCAB = 1000001
TEXT_VOCAB = 10000
EMB = 32
COLOR_LEN = 16
TITLE_LEN = 32

NC = 2
NS = 16
NW = NC * NS
SCHUNK = 1024
NCHUNK = B // SCHUNK
IDXW = 128
L = 16


def _sc_body(item_id_hbm, ctok_hbm, ttok_hbm,
             idtab_hbm, ctab_hbm, ttab_hbm, out_hbm,
             ctab_v, ttab_v, ids_v, idx_v, idg_v,
             ctok_v, ttok_v, outc_v, outt_v, sem):
    w = lax.axis_index("s") * NC + lax.axis_index("c")
    pltpu.sync_copy(ctab_hbm.at[w], ctab_v)
    pltpu.sync_copy(ttab_hbm.at[w], ttab_v)
    comp_base = w * ITEM_VOCAB

    for g in range(NCHUNK):
        s0 = pl.multiple_of(g * SCHUNK, SCHUNK)
        pltpu.sync_copy(item_id_hbm.at[pl.ds(s0, SCHUNK)], ids_v)
        pltpu.sync_copy(ctok_hbm.at[:, pl.ds(s0, SCHUNK)], ctok_v)
        pltpu.sync_copy(ttok_hbm.at[:, pl.ds(s0, SCHUNK)], ttok_v)

        def mk_idx(i, carry):
            sl = pl.ds(i * L, L)
            idx_v[sl] = ids_v[sl] + comp_base
            return carry
        lax.fori_loop(0, SCHUNK // L, mk_idx, 0)

        copies = []
        for j in range(SCHUNK // IDXW):
            copies.append(pltpu.async_copy(
                idtab_hbm.at[idx_v.at[pl.ds(j * IDXW, IDXW)]],
                idg_v.at[pl.ds(j * IDXW, IDXW)], sem))

        def group_body(i, carry):
            sl = pl.ds(i * L, L)
            acc = plsc.load_gather(ctab_v, [ctok_v[0, sl]])
            for t in range(1, COLOR_LEN):
                acc = acc + plsc.load_gather(ctab_v, [ctok_v[t, sl]])
            outc_v[sl] = acc * (1.0 / COLOR_LEN)
            acc2 = plsc.load_gather(ttab_v, [ttok_v[0, sl]])
            for t in range(1, TITLE_LEN):
                acc2 = acc2 + plsc.load_gather(ttab_v, [ttok_v[t, sl]])
            outt_v[sl] = acc2 * (1.0 / TITLE_LEN)
            return carry
        lax.fori_loop(0, SCHUNK // L, group_body, 0)

        for cp in copies:
            cp.wait()

        pltpu.sync_copy(idg_v, out_hbm.at[w, pl.ds(s0, SCHUNK)])
        pltpu.sync_copy(outc_v, out_hbm.at[EMB + w, pl.ds(s0, SCHUNK)])
        pltpu.sync_copy(outt_v, out_hbm.at[2 * EMB + w, pl.ds(s0, SCHUNK)])


@jax.jit
def _sc_call(item_id, ctok_t, ttok_t, idtab_flat, ctab_t, ttab_t):
    f = functools.partial(
        pl.kernel,
        out_type=jax.ShapeDtypeStruct((3 * EMB, B), jnp.float32),
        mesh=plsc.VectorSubcoreMesh(core_axis_name="c", subcore_axis_name="s"),
        scratch_types=[
            pltpu.VMEM((TEXT_VOCAB,), jnp.float32),
            pltpu.VMEM((TEXT_VOCAB,), jnp.float32),
            pltpu.VMEM((SCHUNK,), jnp.int32),
            pltpu.VMEM((SCHUNK,), jnp.int32),
            pltpu.VMEM((SCHUNK,), jnp.float32),
            pltpu.VMEM((COLOR_LEN, SCHUNK), jnp.int32),
            pltpu.VMEM((TITLE_LEN, SCHUNK), jnp.int32),
            pltpu.VMEM((SCHUNK,), jnp.float32),
            pltpu.VMEM((SCHUNK,), jnp.float32),
            pltpu.SemaphoreType.DMA,
        ],
        compiler_params=pltpu.CompilerParams(use_tc_tiling_on_sc=False,
                                             needs_layout_passes=False),
    )(_sc_body)
    return f(item_id, ctok_t, ttok_t, idtab_flat, ctab_t, ttab_t)


def kernel(item_id, color_tokens, title_tokens, id_table, color_table, title_table):
    out_t = _sc_call(item_id,
                     color_tokens.T, title_tokens.T,
                     id_table.T.reshape(-1),
                     color_table.T, title_table.T)
    return out_t.T


# speedup vs baseline: 6.2556x; 6.2556x over previous
import functools

import jax
import jax.numpy as jnp
from jax import lax
from jax.experimental import pallas as pl
from jax.experimental.pallas import tpu as pltpu
from jax.experimental.pallas import tpu_sc as plsc

B = 16384
ITEM_VOCAB = 1000001
TEXT_VOCAB = 10000
EMB = 32
COLOR_LEN = 16
TITLE_LEN = 32

NC = 2
NS = 16
NW = NC * NS
SCHUNK = 1024
NCHUNK = B // SCHUNK
L = 16
IPW = B // NW
NB = 16


def _ct_body(ctok_hbm, ttok_hbm, ctab_hbm, ttab_hbm, out_hbm,
             ctab_v, ttab_v, ctok_v, ttok_v, outc_v, outt_v, sem):
    w = lax.axis_index("s") * NC + lax.axis_index("c")
    pltpu.sync_copy(ctab_hbm.at[w], ctab_v)
    pltpu.sync_copy(ttab_hbm.at[w], ttab_v)

    for g in range(NCHUNK):
        s0 = pl.multiple_of(g * SCHUNK, SCHUNK)
        pltpu.sync_copy(ctok_hbm.at[:, pl.ds(s0, SCHUNK)], ctok_v)
        pltpu.sync_copy(ttok_hbm.at[:, pl.ds(s0, SCHUNK)], ttok_v)

        def group_body(i, carry):
            sl = pl.ds(i * L, L)
            acc = plsc.load_gather(ctab_v, [ctok_v[0, sl]])
            for t in range(1, COLOR_LEN):
                acc = acc + plsc.load_gather(ctab_v, [ctok_v[t, sl]])
            outc_v[sl] = acc * (1.0 / COLOR_LEN)
            acc2 = plsc.load_gather(ttab_v, [ttok_v[0, sl]])
            for t in range(1, TITLE_LEN):
                acc2 = acc2 + plsc.load_gather(ttab_v, [ttok_v[t, sl]])
            outt_v[sl] = acc2 * (1.0 / TITLE_LEN)
            return carry
        lax.fori_loop(0, SCHUNK // L, group_body, 0)

        pltpu.sync_copy(outc_v, out_hbm.at[w, pl.ds(s0, SCHUNK)])
        pltpu.sync_copy(outt_v, out_hbm.at[EMB + w, pl.ds(s0, SCHUNK)])


def _id_body(ids_hbm, tab_hbm, out_hbm, ids_v, bufs_v, obuf_v, sems):
    w = lax.axis_index("s") * NC + lax.axis_index("c")
    base = pl.multiple_of(w * IPW, IPW)
    pltpu.sync_copy(ids_hbm.at[pl.ds(base, IPW)], ids_v)

    def group_body(g, carry):
        j0 = g * NB
        idsvec = ids_v[pl.ds(j0, NB)]
        items = [idsvec[b] for b in range(NB)]
        copies = []
        for b in range(NB):
            blk = pl.multiple_of((items[b] // 8) * 8, 8)
            copies.append(pltpu.async_copy(
                tab_hbm.at[pl.ds(blk, 8), :], bufs_v[b].at[:, :], sems[b]))
        for b in range(NB):
            copies[b].wait()
            r = items[b] % 8
            for h in range(EMB // L):
                sl = pl.ds(h * L, L)
                obuf_v[j0 + b, sl] = bufs_v[b][r, sl]
        return carry
    lax.fori_loop(0, IPW // NB, group_body, 0)

    pltpu.sync_copy(obuf_v, out_hbm.at[pl.ds(base, IPW)])


@jax.jit
def _sc_call(item_id, color_tokens, title_tokens, id_table,
             color_table, title_table):
    mesh = plsc.VectorSubcoreMesh(core_axis_name="c", subcore_axis_name="s")

    ct = functools.partial(
        pl.kernel,
        out_type=jax.ShapeDtypeStruct((2 * EMB, B), jnp.float32),
        mesh=mesh,
        scratch_types=[
            pltpu.VMEM((TEXT_VOCAB,), jnp.float32),
            pltpu.VMEM((TEXT_VOCAB,), jnp.float32),
            pltpu.VMEM((COLOR_LEN, SCHUNK), jnp.int32),
            pltpu.VMEM((TITLE_LEN, SCHUNK), jnp.int32),
            pltpu.VMEM((SCHUNK,), jnp.float32),
            pltpu.VMEM((SCHUNK,), jnp.float32),
            pltpu.SemaphoreType.DMA,
        ],
        compiler_params=pltpu.CompilerParams(use_tc_tiling_on_sc=False,
                                             needs_layout_passes=False),
    )(_ct_body)
    ct_out = ct(color_tokens.T, title_tokens.T, color_table.T, title_table.T)

    idk = functools.partial(
        pl.kernel,
        out_type=jax.ShapeDtypeStruct((B, EMB), jnp.float32),
        mesh=mesh,
        scratch_types=[
            pltpu.VMEM((IPW,), jnp.int32),
            [pltpu.VMEM((8, EMB), jnp.float32) for _ in range(NB)],
            pltpu.VMEM((IPW, EMB), jnp.float32),
            [pltpu.SemaphoreType.DMA for _ in range(NB)],
        ],
        compiler_params=pltpu.CompilerParams(use_tc_tiling_on_sc=True,
                                             needs_layout_passes=False),
    )(_id_body)
    id_out = idk(item_id, id_table)

    return jnp.concatenate([id_out, ct_out.T], axis=1)


def kernel(item_id, color_tokens, title_tokens, id_table, color_table, title_table):
    return _sc_call(item_id, color_tokens, title_tokens,
                    id_table, color_table, title_table)


# speedup vs baseline: 6.8235x; 1.0908x over previous
import functools

import jax
import jax.numpy as jnp
from jax import lax
from jax.experimental import pallas as pl
from jax.experimental.pallas import tpu as pltpu
from jax.experimental.pallas import tpu_sc as plsc

B = 16384
ITEM_VOCAB = 1000001
TEXT_VOCAB = 10000
EMB = 32
COLOR_LEN = 16
TITLE_LEN = 32

NC = 2
NS = 16
NW = NC * NS
SCHUNK = 1024
NCHUNK = B // SCHUNK
L = 16
IPW = B // NW
NB = 16


def _ct_body(ctok_hbm, ttok_hbm, ctab_hbm, ttab_hbm, out_hbm,
             ctab_v, ttab_v, ctok_v, ttok_v, outc_v, outt_v, sem):
    w = lax.axis_index("s") * NC + lax.axis_index("c")
    pltpu.sync_copy(ctab_hbm.at[w], ctab_v)
    pltpu.sync_copy(ttab_hbm.at[w], ttab_v)

    for g in range(NCHUNK):
        s0 = pl.multiple_of(g * SCHUNK, SCHUNK)
        pltpu.sync_copy(ctok_hbm.at[:, pl.ds(s0, SCHUNK)], ctok_v)
        pltpu.sync_copy(ttok_hbm.at[:, pl.ds(s0, SCHUNK)], ttok_v)

        def one_group(i):
            sl = pl.ds(i * L, L)
            p = [plsc.load_gather(ctab_v, [ctok_v[t, sl]]) for t in range(4)]
            for t in range(4, COLOR_LEN):
                p[t % 4] = p[t % 4] + plsc.load_gather(ctab_v, [ctok_v[t, sl]])
            outc_v[sl] = ((p[0] + p[1]) + (p[2] + p[3])) * (1.0 / COLOR_LEN)
            q = [plsc.load_gather(ttab_v, [ttok_v[t, sl]]) for t in range(4)]
            for t in range(4, TITLE_LEN):
                q[t % 4] = q[t % 4] + plsc.load_gather(ttab_v, [ttok_v[t, sl]])
            outt_v[sl] = ((q[0] + q[1]) + (q[2] + q[3])) * (1.0 / TITLE_LEN)

        def group_body(i, carry):
            one_group(2 * i)
            one_group(2 * i + 1)
            return carry
        lax.fori_loop(0, SCHUNK // L // 2, group_body, 0)

        pltpu.sync_copy(outc_v, out_hbm.at[w, pl.ds(s0, SCHUNK)])
        pltpu.sync_copy(outt_v, out_hbm.at[EMB + w, pl.ds(s0, SCHUNK)])


def _id_body(ids_hbm, tab_hbm, out_hbm, ids_v, bufs_v, obuf_v, sems):
    w = lax.axis_index("s") * NC + lax.axis_index("c")
    base = pl.multiple_of(w * IPW, IPW)
    pltpu.sync_copy(ids_hbm.at[pl.ds(base, IPW)], ids_v)
    ngroups = IPW // NB

    def issue(g, h):
        idsvec = ids_v[pl.ds(g * NB, NB)]
        for b in range(NB):
            blk = pl.multiple_of((idsvec[b] // 8) * 8, 8)
            pltpu.async_copy(tab_hbm.at[pl.ds(blk, 8), :],
                             bufs_v[h * NB + b].at[:, :], sems[h])

    def process(g, h):
        j0 = g * NB
        idsvec = ids_v[pl.ds(j0, NB)]
        for b in range(NB):
            pltpu.make_async_copy(tab_hbm.at[pl.ds(0, 8), :],
                                  bufs_v[h * NB + b], sems[h]).wait()
            r = idsvec[b] % 8
            for hh in range(EMB // L):
                sl = pl.ds(hh * L, L)
                obuf_v[j0 + b, sl] = bufs_v[h * NB + b][r, sl]

    issue(0, 0)

    def group_body(k, carry):
        g = 2 * k
        issue(g + 1, 1)
        process(g, 0)

        @pl.when(g + 2 < ngroups)
        def _():
            issue(g + 2, 0)
        process(g + 1, 1)
        return carry
    lax.fori_loop(0, ngroups // 2, group_body, 0)

    pltpu.sync_copy(obuf_v, out_hbm.at[pl.ds(base, IPW)])


@jax.jit
def _sc_call(item_id, color_tokens, title_tokens, id_table,
             color_table, title_table):
    mesh = plsc.VectorSubcoreMesh(core_axis_name="c", subcore_axis_name="s")

    ct = functools.partial(
        pl.kernel,
        out_type=jax.ShapeDtypeStruct((2 * EMB, B), jnp.float32),
        mesh=mesh,
        scratch_types=[
            pltpu.VMEM((TEXT_VOCAB,), jnp.float32),
            pltpu.VMEM((TEXT_VOCAB,), jnp.float32),
            pltpu.VMEM((COLOR_LEN, SCHUNK), jnp.int32),
            pltpu.VMEM((TITLE_LEN, SCHUNK), jnp.int32),
            pltpu.VMEM((SCHUNK,), jnp.float32),
            pltpu.VMEM((SCHUNK,), jnp.float32),
            pltpu.SemaphoreType.DMA,
        ],
        compiler_params=pltpu.CompilerParams(use_tc_tiling_on_sc=False,
                                             needs_layout_passes=False),
    )(_ct_body)
    ct_out = ct(color_tokens.T, title_tokens.T, color_table.T, title_table.T)

    idk = functools.partial(
        pl.kernel,
        out_type=jax.ShapeDtypeStruct((B, EMB), jnp.float32),
        mesh=mesh,
        scratch_types=[
            pltpu.VMEM((IPW,), jnp.int32),
            [pltpu.VMEM((8, EMB), jnp.float32) for _ in range(2 * NB)],
            pltpu.VMEM((IPW, EMB), jnp.float32),
            [pltpu.SemaphoreType.DMA for _ in range(2)],
        ],
        compiler_params=pltpu.CompilerParams(use_tc_tiling_on_sc=True,
                                             needs_layout_passes=False),
    )(_id_body)
    id_out = idk(item_id, id_table)

    return jnp.concatenate([id_out, ct_out.T], axis=1)


def kernel(item_id, color_tokens, title_tokens, id_table, color_table, title_table):
    return _sc_call(item_id, color_tokens, title_tokens,
                    id_table, color_table, title_table)
